# baseline (speedup 1.0000x reference)
import functools

import jax
import jax.numpy as jnp
from jax import lax
from jax.experimental import pallas as pl
from jax.experimental.pallas import tpu as pltpu
from jax.experimental.pallas import tpu_sc as plsc

_E = 16
_TOPK = 2
_BM = 256
_NC = 2
_NS = 16
_NW = _NC * _NS
_L = 16


def _ffn_body(nact_ref, xidx_ref, bmap_ref, x_ref, w1_ref, b1_ref, w2_ref,
              b2_ref, wcol_ref, o_ref):
    g = pl.program_id(0)

    @pl.when(g < nact_ref[0])
    def _():
        hmid = jnp.dot(x_ref[...], w1_ref[0],
                       preferred_element_type=jnp.float32)
        hmid = jnp.maximum(hmid + b1_ref[0], 0.0)
        y = jnp.dot(hmid, w2_ref[0], preferred_element_type=jnp.float32)
        y = y + b2_ref[0]
        o_ref[...] = y * wcol_ref[...]


def _grouped_ffn(nact, xidx, bmap, xs, W1, b1, W2, b2, w_col, NB, P, H, F):
    grid_spec = pltpu.PrefetchScalarGridSpec(
        num_scalar_prefetch=3,
        grid=(NB,),
        in_specs=[
            pl.BlockSpec((_BM, H), lambda g, n, xi, bm: (xi[g], 0)),
            pl.BlockSpec((1, H, F), lambda g, n, xi, bm: (bm[g], 0, 0)),
            pl.BlockSpec((1, 1, F), lambda g, n, xi, bm: (bm[g], 0, 0)),
            pl.BlockSpec((1, F, H), lambda g, n, xi, bm: (bm[g], 0, 0)),
            pl.BlockSpec((1, 1, H), lambda g, n, xi, bm: (bm[g], 0, 0)),
            pl.BlockSpec((_BM, 1), lambda g, n, xi, bm: (xi[g], 0)),
        ],
        out_specs=pl.BlockSpec((_BM, H), lambda g, n, xi, bm: (xi[g], 0)),
    )
    return pl.pallas_call(
        _ffn_body,
        grid_spec=grid_spec,
        out_shape=jax.ShapeDtypeStruct((P, H), jnp.float32),
    )(nact, xidx, bmap, xs, W1, b1, W2, b2, w_col)


def _dispatch(xf, pos, w_flat, tok_in, zeros_i, zeros_f, P, H):
    A = pos.shape[0]
    a_pt = A // _NS
    rows_pw = P // _NW
    CH = 32
    mesh = plsc.VectorSubcoreMesh(core_axis_name="c", subcore_axis_name="s")

    @functools.partial(
        pl.kernel, mesh=mesh,
        out_type=[jax.ShapeDtypeStruct((P, H), jnp.float32),
                  jax.ShapeDtypeStruct((P,), jnp.float32)],
        scratch_types=[
            pltpu.VMEM((a_pt,), jnp.int32),
            pltpu.VMEM((a_pt,), jnp.int32),
            pltpu.VMEM((a_pt,), jnp.float32),
            pltpu.VMEM((rows_pw,), jnp.int32),
            pltpu.VMEM((CH, H), jnp.float32),
            pltpu.VMEM_SHARED((P,), jnp.int32),
            pltpu.VMEM_SHARED((P,), jnp.float32),
            pltpu.SemaphoreType.DMA,
        ],
    )
    def k(xf_h, pos_h, w_h, tok_h, zi_h, zf_h, xs_h, ws_h,
          pos_v, tok_v, w_v, ts_v, rows_v, sh_tok, sh_w, sem):
        s = lax.axis_index("s")
        wid = s * _NC + lax.axis_index("c")

        @pl.when(s == 0)
        def _():
            pltpu.sync_copy(zi_h, sh_tok)
            pltpu.sync_copy(zf_h, sh_w)

        pltpu.sync_copy(pos_h.at[pl.ds(s * a_pt, a_pt)], pos_v)
        pltpu.sync_copy(tok_h.at[pl.ds(s * a_pt, a_pt)], tok_v)
        pltpu.sync_copy(w_h.at[pl.ds(s * a_pt, a_pt)], w_v)
        plsc.subcore_barrier()
        pltpu.sync_copy(tok_v, sh_tok.at[pos_v], add=True)
        pltpu.sync_copy(w_v, sh_w.at[pos_v], add=True)
        plsc.subcore_barrier()

        base = wid * rows_pw
        pltpu.sync_copy(sh_tok.at[pl.ds(base, rows_pw)], ts_v)
        for c in range(rows_pw // CH):
            idx = ts_v.at[pl.ds(c * CH, CH)]
            pltpu.async_copy(xf_h.at[idx], rows_v, sem).wait()
            pltpu.sync_copy(rows_v, xs_h.at[pl.ds(base + c * CH, CH)])
        pltpu.sync_copy(sh_w.at[pl.ds(base, rows_pw)],
                        ws_h.at[pl.ds(base, rows_pw)])

    return k(xf, pos, w_flat, tok_in, zeros_i, zeros_f)


def _combine(ys, p0, p1, T, H):
    tpw = T // _NW
    CH = 32
    mesh = plsc.VectorSubcoreMesh(core_axis_name="c", subcore_axis_name="s")

    @functools.partial(
        pl.kernel, mesh=mesh,
        out_type=jax.ShapeDtypeStruct((T, H), jnp.float32),
        scratch_types=[
            pltpu.VMEM((tpw,), jnp.int32),
            pltpu.VMEM((tpw,), jnp.int32),
            pltpu.VMEM((CH, H), jnp.float32),
            pltpu.VMEM((CH, H), jnp.float32),
            pltpu.SemaphoreType.DMA,
            pltpu.SemaphoreType.DMA,
        ],
    )
    def k(ys_h, p0_h, p1_h, out_h, i0_v, i1_v, ba, bb, s0, s1):
        wid = lax.axis_index("s") * _NC + lax.axis_index("c")
        base = wid * tpw
        pltpu.sync_copy(p0_h.at[pl.ds(base, tpw)], i0_v)
        pltpu.sync_copy(p1_h.at[pl.ds(base, tpw)], i1_v)
        for c in range(tpw // CH):
            ca = pltpu.async_copy(ys_h.at[i0_v.at[pl.ds(c * CH, CH)]], ba, s0)
            cb = pltpu.async_copy(ys_h.at[i1_v.at[pl.ds(c * CH, CH)]], bb, s1)
            ca.wait()
            cb.wait()

            def addrow(r, carry):
                for j in range(H // _L):
                    sl = pl.ds(j * _L, _L)
                    ba[r, sl] = ba[r, sl] + bb[r, sl]
                return carry

            lax.fori_loop(0, CH, addrow, 0)
            pltpu.sync_copy(ba, out_h.at[pl.ds(base + c * CH, CH)])

    return k(ys, p0, p1)


def kernel(x, Wr, br, W1, b1, W2, b2):
    b, s, h = x.shape
    T = b * s
    F = W1.shape[-1]
    E = Wr.shape[-1]
    xf = x.reshape(T, h)

    logits = xf @ Wr + br
    probs = jax.nn.softmax(logits, axis=-1)
    topw, topi = jax.lax.top_k(probs, _TOPK)

    e_flat = topi.T.reshape(-1).astype(jnp.int32)
    w_flat = topw.T.reshape(-1)
    tok = jnp.tile(jnp.arange(T, dtype=jnp.int32), _TOPK)

    onehot = (e_flat[:, None] == jnp.arange(E, dtype=jnp.int32)[None, :]
              ).astype(jnp.int32)
    ranks_all = jnp.cumsum(onehot, axis=0) - onehot
    rank = jnp.sum(ranks_all * onehot, axis=1)
    counts = jnp.sum(onehot, axis=0)
    blocks = (counts + _BM - 1) // _BM
    bstart = jnp.cumsum(blocks) - blocks
    seg_start = _BM * bstart
    pos = seg_start[e_flat] + rank

    NB = (_TOPK * T) // _BM + E
    P = NB * _BM
    nact = jnp.sum(blocks).astype(jnp.int32)

    gidx = jnp.arange(NB, dtype=jnp.int32)
    bmap = jnp.sum(gidx[:, None] >= bstart[None, :], axis=1).astype(
        jnp.int32) - 1
    last = bmap[nact - 1]
    bmap = jnp.where(gidx < nact, bmap, last)
    xidx = jnp.where(gidx < nact, gidx, nact - 1).astype(jnp.int32)

    zeros_i = jnp.zeros((P,), jnp.int32)
    zeros_f = jnp.zeros((P,), jnp.float32)
    xs, w_sorted = _dispatch(xf, pos, w_flat, tok, zeros_i, zeros_f, P, h)

    ys = _grouped_ffn(nact[None], xidx, bmap, xs, W1, b1[:, None, :], W2,
                      b2[:, None, :], w_sorted[:, None], NB, P, h, F)

    out = _combine(ys, pos[:T], pos[T:], T, h)
    return out.reshape(b, s, h)
